# baseline (device time: 671624 ns/iter reference)
import jax
import jax.numpy as jnp
from jax import lax
from jax.experimental import pallas as pl
from jax.experimental.pallas import tpu as pltpu

N_DEV = 4
M_PER = 4096
N_PIECES = 4
P_ROWS = M_PER // N_PIECES
K = 2048
N = 4096
TILE = 512


def kernel(A, B):
    B16 = B.astype(jnp.bfloat16)

    def body(a_ref, b_ref, out_ref, c_ref, a_tile, send_sems, recv_sems,
             copy_sems, a_sems):
        my = lax.axis_index("i")
        left = (my + N_DEV - 1) % N_DEV
        right = (my + 1) % N_DEV

        def rows_of(dev, p):
            return pl.ds(dev * M_PER + p * P_ROWS, P_ROWS)

        def send(src, dev, dst_rows, s_idx, r_idx):
            rdma = pltpu.make_async_remote_copy(
                src_ref=src,
                dst_ref=out_ref.at[dst_rows],
                send_sem=send_sems.at[s_idx],
                recv_sem=recv_sems.at[r_idx],
                device_id=(dev,),
                device_id_type=pl.DeviceIdType.MESH,
            )
            rdma.start()
            return rdma

        def wait_recv(r_idx, dst_rows):
            pltpu.make_async_remote_copy(
                src_ref=c_ref.at[pl.ds(0, P_ROWS)],
                dst_ref=out_ref.at[dst_rows],
                send_sem=send_sems.at[0],
                recv_sem=recv_sems.at[r_idx],
                device_id=(my,),
                device_id_type=pl.DeviceIdType.MESH,
            ).wait_recv()

        n_tiles = M_PER // TILE
        tiles_per_piece = n_tiles // N_PIECES

        def fetch(t):
            pltpu.make_async_copy(
                a_ref.at[pl.ds(t * TILE, TILE)], a_tile.at[t % 2],
                a_sems.at[t % 2],
            ).start()

        def dot_tile(t):
            if t + 1 < n_tiles:
                fetch(t + 1)
            pltpu.make_async_copy(
                a_ref.at[pl.ds(t * TILE, TILE)], a_tile.at[t % 2],
                a_sems.at[t % 2],
            ).wait()
            c_ref[pl.ds(t * TILE, TILE), :] = jnp.dot(
                a_tile[t % 2].astype(jnp.bfloat16), b_ref[:, :],
                preferred_element_type=jnp.float32,
            ).astype(jnp.bfloat16)

        fetch(0)

        bar = pltpu.get_barrier_semaphore()
        for nbr in (left, right):
            pl.semaphore_signal(
                bar, inc=1, device_id=(nbr,),
                device_id_type=pl.DeviceIdType.MESH,
            )
        pl.semaphore_wait(bar, 2)

        sends = []
        locals_ = []
        for p in range(N_PIECES):
            for t in range(p * tiles_per_piece, (p + 1) * tiles_per_piece):
                dot_tile(t)
            piece = c_ref.at[pl.ds(p * P_ROWS, P_ROWS)]
            cp = pltpu.make_async_copy(
                piece, out_ref.at[rows_of(my, p)], copy_sems.at[p])
            cp.start()
            locals_.append(cp)
            sends.append(send(piece, right, rows_of(my, p), 2 * p, p))
            sends.append(send(piece, left, rows_of(my, p), 2 * p + 1, 4 + p))

        for p in (0, 1):
            wait_recv(p, rows_of(left, p))
            sends.append(
                send(out_ref.at[rows_of(left, p)], right, rows_of(left, p),
                     8 + p, 8 + p))
        for p in (2, 3):
            wait_recv(4 + p, rows_of(right, p))
            sends.append(
                send(out_ref.at[rows_of(right, p)], left, rows_of(right, p),
                     10 + p - 2, 10 + p - 2))

        diag = (my + 2) % N_DEV
        for p in (2, 3):
            wait_recv(p, rows_of(left, p))
        for p in (0, 1):
            wait_recv(4 + p, rows_of(right, p))
        for p in range(N_PIECES):
            wait_recv(8 + p, rows_of(diag, p))
        for cp in locals_:
            cp.wait()
        for rdma in sends:
            rdma.wait_send()

    return pl.pallas_call(
        body,
        out_shape=jax.ShapeDtypeStruct((N_DEV * M_PER, N), jnp.bfloat16),
        in_specs=[
            pl.BlockSpec(memory_space=pl.ANY),
            pl.BlockSpec(memory_space=pltpu.VMEM),
        ],
        out_specs=pl.BlockSpec(memory_space=pl.ANY),
        scratch_shapes=[
            pltpu.VMEM((M_PER, N), jnp.bfloat16),
            pltpu.VMEM((2, TILE, K), jnp.float32),
            pltpu.SemaphoreType.DMA((12,)),
            pltpu.SemaphoreType.DMA((12,)),
            pltpu.SemaphoreType.DMA((N_PIECES,)),
            pltpu.SemaphoreType.DMA((2,)),
        ],
        compiler_params=pltpu.CompilerParams(
            collective_id=0,
            vmem_limit_bytes=100 * 1024 * 1024,
        ),
    )(A, B16)


# device time: 662646 ns/iter; 1.0135x vs baseline; 1.0135x over previous
import jax
import jax.numpy as jnp
from jax import lax
from jax.experimental import pallas as pl
from jax.experimental.pallas import tpu as pltpu

N_DEV = 4
M_PER = 4096
N_PIECES = 8
P_ROWS = M_PER // N_PIECES
K = 2048
N = 4096
TILE = 512
NP = N_PIECES


def kernel(A, B):
    B16 = B.astype(jnp.bfloat16)

    def body(a_ref, b_ref, out_ref, c_ref, a_tile, send_sems, recv_sems,
             copy_sems, a_sems):
        my = lax.axis_index("i")
        left = (my + N_DEV - 1) % N_DEV
        right = (my + 1) % N_DEV

        def rows_of(dev, p):
            return pl.ds(dev * M_PER + p * P_ROWS, P_ROWS)

        def send(src, dev, dst_rows, s_idx, r_idx):
            rdma = pltpu.make_async_remote_copy(
                src_ref=src,
                dst_ref=out_ref.at[dst_rows],
                send_sem=send_sems.at[s_idx],
                recv_sem=recv_sems.at[r_idx],
                device_id=(dev,),
                device_id_type=pl.DeviceIdType.MESH,
            )
            rdma.start()
            return rdma

        def wait_recv(r_idx, dst_rows):
            pltpu.make_async_remote_copy(
                src_ref=c_ref.at[pl.ds(0, P_ROWS)],
                dst_ref=out_ref.at[dst_rows],
                send_sem=send_sems.at[0],
                recv_sem=recv_sems.at[r_idx],
                device_id=(my,),
                device_id_type=pl.DeviceIdType.MESH,
            ).wait_recv()

        n_tiles = M_PER // TILE
        tiles_per_piece = n_tiles // N_PIECES

        def fetch(t):
            pltpu.make_async_copy(
                a_ref.at[pl.ds(t * TILE, TILE)], a_tile.at[t % 2],
                a_sems.at[t % 2],
            ).start()

        def dot_tile(t):
            if t + 1 < n_tiles:
                fetch(t + 1)
            pltpu.make_async_copy(
                a_ref.at[pl.ds(t * TILE, TILE)], a_tile.at[t % 2],
                a_sems.at[t % 2],
            ).wait()
            c_ref[pl.ds(t * TILE, TILE), :] = jnp.dot(
                a_tile[t % 2].astype(jnp.bfloat16), b_ref[:, :],
                preferred_element_type=jnp.float32,
            ).astype(jnp.bfloat16)

        fetch(0)

        bar = pltpu.get_barrier_semaphore()
        for nbr in (left, right):
            pl.semaphore_signal(
                bar, inc=1, device_id=(nbr,),
                device_id_type=pl.DeviceIdType.MESH,
            )
        pl.semaphore_wait(bar, 2)

        sends = []
        locals_ = []
        for p in range(N_PIECES):
            for t in range(p * tiles_per_piece, (p + 1) * tiles_per_piece):
                dot_tile(t)
            piece = c_ref.at[pl.ds(p * P_ROWS, P_ROWS)]
            cp = pltpu.make_async_copy(
                piece, out_ref.at[rows_of(my, p)], copy_sems.at[p])
            cp.start()
            locals_.append(cp)
            sends.append(send(piece, right, rows_of(my, p), 2 * p, p))
            sends.append(send(piece, left, rows_of(my, p), 2 * p + 1, NP + p))

        for p in range(NP // 2):
            wait_recv(p, rows_of(left, p))
            sends.append(
                send(out_ref.at[rows_of(left, p)], right, rows_of(left, p),
                     2 * NP + p, 2 * NP + p))
        for p in range(NP // 2, NP):
            wait_recv(NP + p, rows_of(right, p))
            sends.append(
                send(out_ref.at[rows_of(right, p)], left, rows_of(right, p),
                     2 * NP + p, 2 * NP + p))

        diag = (my + 2) % N_DEV
        for p in range(NP // 2, NP):
            wait_recv(p, rows_of(left, p))
        for p in range(NP // 2):
            wait_recv(NP + p, rows_of(right, p))
        for p in range(NP):
            wait_recv(2 * NP + p, rows_of(diag, p))
        for cp in locals_:
            cp.wait()
        for rdma in sends:
            rdma.wait_send()

    return pl.pallas_call(
        body,
        out_shape=jax.ShapeDtypeStruct((N_DEV * M_PER, N), jnp.bfloat16),
        in_specs=[
            pl.BlockSpec(memory_space=pl.ANY),
            pl.BlockSpec(memory_space=pltpu.VMEM),
        ],
        out_specs=pl.BlockSpec(memory_space=pl.ANY),
        scratch_shapes=[
            pltpu.VMEM((M_PER, N), jnp.bfloat16),
            pltpu.VMEM((2, TILE, K), jnp.float32),
            pltpu.SemaphoreType.DMA((3 * NP,)),
            pltpu.SemaphoreType.DMA((3 * NP,)),
            pltpu.SemaphoreType.DMA((N_PIECES,)),
            pltpu.SemaphoreType.DMA((2,)),
        ],
        compiler_params=pltpu.CompilerParams(
            collective_id=0,
            vmem_limit_bytes=100 * 1024 * 1024,
        ),
    )(A, B16)
